# Initial kernel scaffold; baseline (speedup 1.0000x reference)
import jax
import jax.numpy as jnp
from jax.experimental import pallas as pl


def kernel(adj, features, W1, b1, W2, b2, W3, b3, W4, b4, att_W, fc1_W, fc1_b, sc_W, sc_b):
    raise NotImplementedError("write your pallas kernel here")


# speedup vs baseline: 12.0241x; 12.0241x over previous
import functools

import jax
import jax.numpy as jnp
from jax import lax
from jax.experimental import pallas as pl
from jax.experimental.pallas import tpu as pltpu
from jax.experimental.pallas import tpu_sc as plsc

NC = 2
NS = 16
NW = NC * NS


def _make_edge_scatter(n, e, f, chunk):
    assert e % NW == 0
    epw = e // NW
    assert epw % chunk == 0 and chunk % 8 == 0 and chunk <= 128
    nchunks = epw // chunk
    rpt = (n // NS) & ~7
    rem = n - NS * rpt

    mesh = plsc.VectorSubcoreMesh(core_axis_name="c", subcore_axis_name="s")

    @functools.partial(
        pl.kernel,
        out_type=jax.ShapeDtypeStruct((NC, n, f), jnp.float32),
        mesh=mesh,
        scratch_types=[
            pltpu.VMEM((2, chunk), jnp.int32),
            pltpu.VMEM((2, chunk), jnp.int32),
            pltpu.VMEM((2, chunk, f), jnp.float32),
            pltpu.VMEM_SHARED((n, f), jnp.float32),
            pltpu.SemaphoreType.DMA,
        ],
        compiler_params=pltpu.CompilerParams(use_tc_tiling_on_sc=False),
    )
    def k(src_hbm, dst_hbm, h_hbm, zero_hbm, out_hbm, sidx, didx, rows, acc,
          sem):
        c = lax.axis_index("c")
        s = lax.axis_index("s")
        wid = s * NC + c
        base = wid * epw

        pltpu.sync_copy(zero_hbm.at[pl.ds(s * rpt, rpt)],
                        acc.at[pl.ds(s * rpt, rpt)])
        if rem:
            @pl.when(s == NS - 1)
            def _():
                pltpu.sync_copy(zero_hbm.at[pl.ds(NS * rpt, rem)],
                                acc.at[pl.ds(NS * rpt, rem)])
        plsc.subcore_barrier()

        def chunk_body(j, carry):
            jb = lax.rem(j, 2)
            off = base + j * chunk
            pltpu.sync_copy(src_hbm.at[pl.ds(off, chunk)], sidx.at[jb])
            pltpu.sync_copy(dst_hbm.at[pl.ds(off, chunk)], didx.at[jb])
            pltpu.async_copy(h_hbm.at[sidx.at[jb]], rows.at[jb], sem).wait()
            pltpu.sync_copy(rows.at[jb], acc.at[didx.at[jb]], add=True)
            return carry

        lax.fori_loop(0, nchunks, chunk_body, 0)
        plsc.subcore_barrier()
        pltpu.sync_copy(acc.at[pl.ds(s * rpt, rpt)],
                        out_hbm.at[c, pl.ds(s * rpt, rpt)])
        if rem:
            @pl.when(s == NS - 1)
            def _():
                pltpu.sync_copy(acc.at[pl.ds(NS * rpt, rem)],
                                out_hbm.at[c, pl.ds(NS * rpt, rem)])

    return k


def _make_degree(n, e, chunk):
    assert e % NW == 0
    epw = e // NW
    assert epw % chunk == 0 and chunk % 8 == 0 and chunk <= 128
    nchunks = epw // chunk
    rpt = (n // NS) & ~7
    rem = n - NS * rpt
    f = 16

    mesh = plsc.VectorSubcoreMesh(core_axis_name="c", subcore_axis_name="s")

    @functools.partial(
        pl.kernel,
        out_type=jax.ShapeDtypeStruct((NC, n, f), jnp.float32),
        mesh=mesh,
        scratch_types=[
            pltpu.VMEM((2, chunk), jnp.int32),
            pltpu.VMEM((chunk, f), jnp.float32),
            pltpu.VMEM_SHARED((n, f), jnp.float32),
        ],
        compiler_params=pltpu.CompilerParams(use_tc_tiling_on_sc=False),
    )
    def k(dst_hbm, ones_hbm, zero_hbm, out_hbm, didx, ones_v, acc):
        c = lax.axis_index("c")
        s = lax.axis_index("s")
        wid = s * NC + c
        base = wid * epw

        pltpu.sync_copy(ones_hbm, ones_v)
        pltpu.sync_copy(zero_hbm.at[pl.ds(s * rpt, rpt)],
                        acc.at[pl.ds(s * rpt, rpt)])
        if rem:
            @pl.when(s == NS - 1)
            def _():
                pltpu.sync_copy(zero_hbm.at[pl.ds(NS * rpt, rem)],
                                acc.at[pl.ds(NS * rpt, rem)])
        plsc.subcore_barrier()

        def chunk_body(j, carry):
            jb = lax.rem(j, 2)
            off = base + j * chunk
            pltpu.sync_copy(dst_hbm.at[pl.ds(off, chunk)], didx.at[jb])
            pltpu.sync_copy(ones_v, acc.at[didx.at[jb]], add=True)
            return carry

        lax.fori_loop(0, nchunks, chunk_body, 0)
        plsc.subcore_barrier()
        pltpu.sync_copy(acc.at[pl.ds(s * rpt, rpt)],
                        out_hbm.at[c, pl.ds(s * rpt, rpt)])
        if rem:
            @pl.when(s == NS - 1)
            def _():
                pltpu.sync_copy(acc.at[pl.ds(NS * rpt, rem)],
                                out_hbm.at[c, pl.ds(NS * rpt, rem)])

    return k


def _tc_call(body, out_shapes):
    return pl.pallas_call(body, out_shape=out_shapes)


def _k_first(degp_ref, x_ref, w_ref, dinv_ref, gs_ref):
    deg = 1.0 + degp_ref[0, :, 0] + degp_ref[1, :, 0]
    dinv = lax.rsqrt(deg)
    dinv_ref[...] = dinv[:, None]
    h = jnp.dot(x_ref[...], w_ref[...], preferred_element_type=jnp.float32)
    gs_ref[...] = dinv[:, None] * h


def _k_mid(p_ref, gs_ref, b_ref, dinv_ref, w_ref, gsn_ref):
    dinv = dinv_ref[...]
    y = dinv * (p_ref[0] + p_ref[1] + gs_ref[...]) + b_ref[...][None, :]
    y = jnp.maximum(y, 0.0)
    h = jnp.dot(y, w_ref[...], preferred_element_type=jnp.float32)
    gsn_ref[...] = dinv * h


def _k_last(p_ref, gs_ref, b_ref, dinv_ref, attw_ref, fc1w_ref, fc1b_ref,
            scw_ref, scb_ref, out_ref):
    dinv = dinv_ref[...]
    h = dinv * (p_ref[0] + p_ref[1] + gs_ref[...]) + b_ref[...][None, :]
    h = jnp.maximum(h, 0.0)
    n = h.shape[0]
    hw = jnp.dot(h, attw_ref[...], preferred_element_type=jnp.float32)
    gc = jnp.sum(hw, axis=0) * (1.0 / n)
    tg = jnp.tanh(gc)
    scores = jax.nn.sigmoid(
        jnp.sum(h * tg[None, :], axis=1, keepdims=True))
    rep = jnp.sum(h * scores, axis=0)[None, :]
    s = jnp.dot(rep, fc1w_ref[...], preferred_element_type=jnp.float32)
    s = jnp.maximum(s + fc1b_ref[...][None, :], 0.0)
    s = jnp.dot(s, scw_ref[...], preferred_element_type=jnp.float32)
    s = s + scb_ref[...][None, :]
    m = jnp.max(s, axis=1, keepdims=True)
    out_ref[...] = s - m - jnp.log(
        jnp.sum(jnp.exp(s - m), axis=1, keepdims=True))


def kernel(adj, features, W1, b1, W2, b2, W3, b3, W4, b4, att_W, fc1_W,
           fc1_b, sc_W, sc_b):
    n, d = features.shape
    e = adj.shape[1]
    chunk = 80
    f32 = jnp.float32
    src = adj[0]
    dst = adj[1]

    deg_k = _make_degree(n, e, chunk)
    degp = deg_k(dst, jnp.ones((chunk, 16), f32), jnp.zeros((n, 16), f32))

    dinv, gs = _tc_call(
        _k_first,
        (jax.ShapeDtypeStruct((n, 1), f32),
         jax.ShapeDtypeStruct((n, W1.shape[1]), f32)),
    )(degp, features, W1)

    weights = [(b1, W2), (b2, W3), (b3, W4)]
    for b, wn in weights:
        f = gs.shape[1]
        scat = _make_edge_scatter(n, e, f, chunk)
        p = scat(src, dst, gs, jnp.zeros((n, f), f32))
        gs = _tc_call(
            _k_mid, jax.ShapeDtypeStruct((n, wn.shape[1]), f32)
        )(p, gs, b, dinv, wn)

    f = gs.shape[1]
    scat = _make_edge_scatter(n, e, f, chunk)
    p = scat(src, dst, gs, jnp.zeros((n, f), f32))
    out = _tc_call(
        _k_last, jax.ShapeDtypeStruct((1, sc_W.shape[1]), f32)
    )(p, gs, b4, dinv, att_W, fc1_W, fc1_b, sc_W, sc_b)
    return out


# speedup vs baseline: 40.8536x; 3.3976x over previous
import functools

import jax
import jax.numpy as jnp
from jax import lax
from jax.experimental import pallas as pl
from jax.experimental.pallas import tpu as pltpu
from jax.experimental.pallas import tpu_sc as plsc

NC = 2
NS = 16
NW = NC * NS


def _make_edge_scatter(n, e, f, chunk):
    assert e % NW == 0
    epw = e // NW
    assert epw % chunk == 0 and chunk % 8 == 0 and chunk <= 128
    nchunks = epw // chunk
    nbuf = 5
    assert nchunks % nbuf == 0
    ngroups = nchunks // nbuf
    rpt = (n // NS) & ~7
    rem = n - NS * rpt

    mesh = plsc.VectorSubcoreMesh(core_axis_name="c", subcore_axis_name="s")

    @functools.partial(
        pl.kernel,
        out_type=jax.ShapeDtypeStruct((NC, n, f), jnp.float32),
        mesh=mesh,
        scratch_types=[
            pltpu.VMEM((nchunks, chunk), jnp.int32),
            pltpu.VMEM((nchunks, chunk), jnp.int32),
            pltpu.VMEM((nbuf, chunk, f), jnp.float32),
            pltpu.VMEM_SHARED((n, f), jnp.float32),
            pltpu.SemaphoreType.DMA,
        ],
        compiler_params=pltpu.CompilerParams(use_tc_tiling_on_sc=False),
    )
    def k(src_hbm, dst_hbm, h_hbm, zero_hbm, out_hbm, sidx, didx, rows, acc,
          gsem):
        c = lax.axis_index("c")
        s = lax.axis_index("s")
        wid = s * NC + c

        pltpu.sync_copy(src_hbm.at[wid], sidx)
        pltpu.sync_copy(dst_hbm.at[wid], didx)

        pltpu.sync_copy(zero_hbm.at[pl.ds(s * rpt, rpt)],
                        acc.at[pl.ds(s * rpt, rpt)])
        if rem:
            @pl.when(s == NS - 1)
            def _():
                pltpu.sync_copy(zero_hbm.at[pl.ds(NS * rpt, rem)],
                                acc.at[pl.ds(NS * rpt, rem)])
        plsc.subcore_barrier()

        for b in range(nbuf):
            pltpu.async_copy(h_hbm.at[sidx.at[b]], rows.at[b], gsem)

        def group_body(g, carry):
            j0 = g * nbuf
            for b in range(nbuf):
                pltpu.make_async_copy(h_hbm.at[sidx.at[j0 + b]],
                                      rows.at[b], gsem).wait()
                pltpu.sync_copy(rows.at[b], acc.at[didx.at[j0 + b]],
                                add=True)

                @pl.when(j0 + b + nbuf < nchunks)
                def _():
                    pltpu.async_copy(h_hbm.at[sidx.at[j0 + b + nbuf]],
                                     rows.at[b], gsem)
            return carry

        lax.fori_loop(0, ngroups, group_body, 0)
        plsc.subcore_barrier()
        pltpu.sync_copy(acc.at[pl.ds(s * rpt, rpt)],
                        out_hbm.at[c, pl.ds(s * rpt, rpt)])
        if rem:
            @pl.when(s == NS - 1)
            def _():
                pltpu.sync_copy(acc.at[pl.ds(NS * rpt, rem)],
                                out_hbm.at[c, pl.ds(NS * rpt, rem)])

    return k


def _make_degree(n, e, chunk):
    assert e % NW == 0
    epw = e // NW
    assert epw % chunk == 0 and chunk % 8 == 0 and chunk <= 128
    nchunks = epw // chunk
    rpt = (n // NS) & ~7
    rem = n - NS * rpt
    f = 16

    mesh = plsc.VectorSubcoreMesh(core_axis_name="c", subcore_axis_name="s")

    @functools.partial(
        pl.kernel,
        out_type=jax.ShapeDtypeStruct((NC, n, f), jnp.float32),
        mesh=mesh,
        scratch_types=[
            pltpu.VMEM((nchunks, chunk), jnp.int32),
            pltpu.VMEM((chunk, f), jnp.float32),
            pltpu.VMEM_SHARED((n, f), jnp.float32),
        ],
        compiler_params=pltpu.CompilerParams(use_tc_tiling_on_sc=False),
    )
    def k(dst_hbm, ones_hbm, zero_hbm, out_hbm, didx, ones_v, acc):
        c = lax.axis_index("c")
        s = lax.axis_index("s")
        wid = s * NC + c

        pltpu.sync_copy(dst_hbm.at[wid], didx)
        pltpu.sync_copy(ones_hbm, ones_v)
        pltpu.sync_copy(zero_hbm.at[pl.ds(s * rpt, rpt)],
                        acc.at[pl.ds(s * rpt, rpt)])
        if rem:
            @pl.when(s == NS - 1)
            def _():
                pltpu.sync_copy(zero_hbm.at[pl.ds(NS * rpt, rem)],
                                acc.at[pl.ds(NS * rpt, rem)])
        plsc.subcore_barrier()

        def chunk_body(j, carry):
            pltpu.sync_copy(ones_v, acc.at[didx.at[j]], add=True)
            return carry

        lax.fori_loop(0, nchunks, chunk_body, 0)
        plsc.subcore_barrier()
        pltpu.sync_copy(acc.at[pl.ds(s * rpt, rpt)],
                        out_hbm.at[c, pl.ds(s * rpt, rpt)])
        if rem:
            @pl.when(s == NS - 1)
            def _():
                pltpu.sync_copy(acc.at[pl.ds(NS * rpt, rem)],
                                out_hbm.at[c, pl.ds(NS * rpt, rem)])

    return k


def _tc_call(body, out_shapes):
    return pl.pallas_call(body, out_shape=out_shapes)


def _k_first(degp_ref, x_ref, w_ref, dinv_ref, gs_ref):
    deg = 1.0 + degp_ref[0, :, 0] + degp_ref[1, :, 0]
    dinv = lax.rsqrt(deg)
    dinv_ref[...] = dinv[:, None]
    h = jnp.dot(x_ref[...], w_ref[...], preferred_element_type=jnp.float32)
    gs_ref[...] = dinv[:, None] * h


def _k_mid(p_ref, gs_ref, b_ref, dinv_ref, w_ref, gsn_ref):
    dinv = dinv_ref[...]
    y = dinv * (p_ref[0] + p_ref[1] + gs_ref[...]) + b_ref[...][None, :]
    y = jnp.maximum(y, 0.0)
    h = jnp.dot(y, w_ref[...], preferred_element_type=jnp.float32)
    gsn_ref[...] = dinv * h


def _k_last(p_ref, gs_ref, b_ref, dinv_ref, attw_ref, fc1w_ref, fc1b_ref,
            scw_ref, scb_ref, out_ref):
    dinv = dinv_ref[...]
    h = dinv * (p_ref[0] + p_ref[1] + gs_ref[...]) + b_ref[...][None, :]
    h = jnp.maximum(h, 0.0)
    n = h.shape[0]
    hw = jnp.dot(h, attw_ref[...], preferred_element_type=jnp.float32)
    gc = jnp.sum(hw, axis=0) * (1.0 / n)
    tg = jnp.tanh(gc)
    scores = jax.nn.sigmoid(
        jnp.sum(h * tg[None, :], axis=1, keepdims=True))
    rep = jnp.sum(h * scores, axis=0)[None, :]
    s = jnp.dot(rep, fc1w_ref[...], preferred_element_type=jnp.float32)
    s = jnp.maximum(s + fc1b_ref[...][None, :], 0.0)
    s = jnp.dot(s, scw_ref[...], preferred_element_type=jnp.float32)
    s = s + scb_ref[...][None, :]
    m = jnp.max(s, axis=1, keepdims=True)
    out_ref[...] = s - m - jnp.log(
        jnp.sum(jnp.exp(s - m), axis=1, keepdims=True))


def kernel(adj, features, W1, b1, W2, b2, W3, b3, W4, b4, att_W, fc1_W,
           fc1_b, sc_W, sc_b):
    n, d = features.shape
    e = adj.shape[1]
    chunk = 80
    f32 = jnp.float32
    epw = e // NW
    src = adj[0].reshape(NW, epw // chunk, chunk)
    dst = adj[1].reshape(NW, epw // chunk, chunk)

    deg_k = _make_degree(n, e, chunk)
    degp = deg_k(dst, jnp.ones((chunk, 16), f32), jnp.zeros((n, 16), f32))

    dinv, gs = _tc_call(
        _k_first,
        (jax.ShapeDtypeStruct((n, 1), f32),
         jax.ShapeDtypeStruct((n, W1.shape[1]), f32)),
    )(degp, features, W1)

    weights = [(b1, W2), (b2, W3), (b3, W4)]
    for b, wn in weights:
        f = gs.shape[1]
        scat = _make_edge_scatter(n, e, f, chunk)
        p = scat(src, dst, gs, jnp.zeros((n, f), f32))
        gs = _tc_call(
            _k_mid, jax.ShapeDtypeStruct((n, wn.shape[1]), f32)
        )(p, gs, b, dinv, wn)

    f = gs.shape[1]
    scat = _make_edge_scatter(n, e, f, chunk)
    p = scat(src, dst, gs, jnp.zeros((n, f), f32))
    out = _tc_call(
        _k_last, jax.ShapeDtypeStruct((1, sc_W.shape[1]), f32)
    )(p, gs, b4, dinv, att_W, fc1_W, fc1_b, sc_W, sc_b)
    return out


# speedup vs baseline: 42.1739x; 1.0323x over previous
import functools

import jax
import jax.numpy as jnp
from jax import lax
from jax.experimental import pallas as pl
from jax.experimental.pallas import tpu as pltpu
from jax.experimental.pallas import tpu_sc as plsc

NC = 2
NS = 16
NW = NC * NS


def _make_edge_scatter(n, e, f, chunk):
    assert e % NW == 0
    epw = e // NW
    assert epw % chunk == 0 and chunk % 8 == 0 and chunk <= 128
    nchunks = epw // chunk
    nbuf = 5
    nring = 2 * nbuf
    rpt = (n // NS) & ~7
    rem = n - NS * rpt

    mesh = plsc.VectorSubcoreMesh(core_axis_name="c", subcore_axis_name="s")

    @functools.partial(
        pl.kernel,
        out_type=jax.ShapeDtypeStruct((NC, n, f), jnp.float32),
        mesh=mesh,
        scratch_types=[
            pltpu.VMEM((nchunks, chunk), jnp.int32),
            pltpu.VMEM((nchunks, chunk), jnp.int32),
            pltpu.VMEM((nring, chunk, f), jnp.float32),
            pltpu.VMEM_SHARED((n, f), jnp.float32),
            pltpu.SemaphoreType.DMA,
            pltpu.SemaphoreType.DMA,
        ],
        compiler_params=pltpu.CompilerParams(use_tc_tiling_on_sc=False),
    )
    def k(src_hbm, dst_hbm, h_hbm, zero_hbm, out_hbm, sidx, didx, rows, acc,
          gsem, ssem):
        c = lax.axis_index("c")
        s = lax.axis_index("s")
        wid = s * NC + c

        pltpu.sync_copy(src_hbm.at[wid], sidx)
        pltpu.sync_copy(dst_hbm.at[wid], didx)

        pltpu.sync_copy(zero_hbm.at[pl.ds(s * rpt, rpt)],
                        acc.at[pl.ds(s * rpt, rpt)])
        if rem:
            @pl.when(s == NS - 1)
            def _():
                pltpu.sync_copy(zero_hbm.at[pl.ds(NS * rpt, rem)],
                                acc.at[pl.ds(NS * rpt, rem)])
        plsc.subcore_barrier()

        for b in range(nbuf):
            pltpu.async_copy(h_hbm.at[sidx.at[b]], rows.at[b], gsem)

        def chunk_body(j, carry):
            slot = lax.rem(j, nring)
            pltpu.make_async_copy(h_hbm.at[sidx.at[j]], rows.at[slot],
                                  gsem).wait()
            pltpu.async_copy(rows.at[slot], acc.at[didx.at[j]], ssem,
                             add=True)

            @pl.when(j >= nbuf)
            def _():
                pltpu.make_async_copy(rows.at[slot], acc.at[didx.at[j]],
                                      ssem).wait()

            @pl.when(j + nbuf < nchunks)
            def _():
                pltpu.async_copy(h_hbm.at[sidx.at[j + nbuf]],
                                 rows.at[lax.rem(j + nbuf, nring)], gsem)
            return carry

        lax.fori_loop(0, nchunks, chunk_body, 0)
        for b in range(nbuf):
            pltpu.make_async_copy(rows.at[0], acc.at[didx.at[0]],
                                  ssem).wait()
        plsc.subcore_barrier()
        pltpu.sync_copy(acc.at[pl.ds(s * rpt, rpt)],
                        out_hbm.at[c, pl.ds(s * rpt, rpt)])
        if rem:
            @pl.when(s == NS - 1)
            def _():
                pltpu.sync_copy(acc.at[pl.ds(NS * rpt, rem)],
                                out_hbm.at[c, pl.ds(NS * rpt, rem)])

    return k


def _make_degree(n, e, chunk):
    assert e % NW == 0
    epw = e // NW
    assert epw % chunk == 0 and chunk % 8 == 0 and chunk <= 128
    nchunks = epw // chunk
    rpt = (n // NS) & ~7
    rem = n - NS * rpt
    f = 16

    mesh = plsc.VectorSubcoreMesh(core_axis_name="c", subcore_axis_name="s")

    @functools.partial(
        pl.kernel,
        out_type=jax.ShapeDtypeStruct((NC, n, f), jnp.float32),
        mesh=mesh,
        scratch_types=[
            pltpu.VMEM((nchunks, chunk), jnp.int32),
            pltpu.VMEM((chunk, f), jnp.float32),
            pltpu.VMEM_SHARED((n, f), jnp.float32),
            pltpu.SemaphoreType.DMA,
        ],
        compiler_params=pltpu.CompilerParams(use_tc_tiling_on_sc=False),
    )
    def k(dst_hbm, ones_hbm, zero_hbm, out_hbm, didx, ones_v, acc, ssem):
        c = lax.axis_index("c")
        s = lax.axis_index("s")
        wid = s * NC + c

        pltpu.sync_copy(dst_hbm.at[wid], didx)
        pltpu.sync_copy(ones_hbm, ones_v)
        pltpu.sync_copy(zero_hbm.at[pl.ds(s * rpt, rpt)],
                        acc.at[pl.ds(s * rpt, rpt)])
        if rem:
            @pl.when(s == NS - 1)
            def _():
                pltpu.sync_copy(zero_hbm.at[pl.ds(NS * rpt, rem)],
                                acc.at[pl.ds(NS * rpt, rem)])
        plsc.subcore_barrier()

        lag = 8

        def chunk_body(j, carry):
            pltpu.async_copy(ones_v, acc.at[didx.at[j]], ssem, add=True)

            @pl.when(j >= lag)
            def _():
                pltpu.make_async_copy(ones_v, acc.at[didx.at[j]],
                                      ssem).wait()
            return carry

        lax.fori_loop(0, nchunks, chunk_body, 0)
        for _ in range(min(lag, nchunks)):
            pltpu.make_async_copy(ones_v, acc.at[didx.at[0]],
                                  ssem).wait()
        plsc.subcore_barrier()
        pltpu.sync_copy(acc.at[pl.ds(s * rpt, rpt)],
                        out_hbm.at[c, pl.ds(s * rpt, rpt)])
        if rem:
            @pl.when(s == NS - 1)
            def _():
                pltpu.sync_copy(acc.at[pl.ds(NS * rpt, rem)],
                                out_hbm.at[c, pl.ds(NS * rpt, rem)])

    return k


def _tc_call(body, out_shapes):
    return pl.pallas_call(body, out_shape=out_shapes)


def _k_matmul(x_ref, w_ref, g_ref):
    g_ref[...] = jnp.dot(x_ref[...], w_ref[...],
                         preferred_element_type=jnp.float32)


def _k_scale(degp_ref, g_ref, dinv_ref, gs_ref):
    deg = 1.0 + degp_ref[0, :, 0] + degp_ref[1, :, 0]
    dinv = lax.rsqrt(deg)
    dinv_ref[...] = dinv[:, None]
    gs_ref[...] = dinv[:, None] * g_ref[...]


def _k_mid(p_ref, gs_ref, b_ref, dinv_ref, w_ref, gsn_ref):
    dinv = dinv_ref[...]
    y = dinv * (p_ref[0] + p_ref[1] + gs_ref[...]) + b_ref[...][None, :]
    y = jnp.maximum(y, 0.0)
    h = jnp.dot(y, w_ref[...], preferred_element_type=jnp.float32)
    gsn_ref[...] = dinv * h


def _k_last(p_ref, gs_ref, b_ref, dinv_ref, attw_ref, fc1w_ref, fc1b_ref,
            scw_ref, scb_ref, out_ref):
    dinv = dinv_ref[...]
    h = dinv * (p_ref[0] + p_ref[1] + gs_ref[...]) + b_ref[...][None, :]
    h = jnp.maximum(h, 0.0)
    n = h.shape[0]
    hw = jnp.dot(h, attw_ref[...], preferred_element_type=jnp.float32)
    gc = jnp.sum(hw, axis=0) * (1.0 / n)
    tg = jnp.tanh(gc)
    scores = jax.nn.sigmoid(
        jnp.sum(h * tg[None, :], axis=1, keepdims=True))
    rep = jnp.sum(h * scores, axis=0)[None, :]
    s = jnp.dot(rep, fc1w_ref[...], preferred_element_type=jnp.float32)
    s = jnp.maximum(s + fc1b_ref[...][None, :], 0.0)
    s = jnp.dot(s, scw_ref[...], preferred_element_type=jnp.float32)
    s = s + scb_ref[...][None, :]
    m = jnp.max(s, axis=1, keepdims=True)
    out_ref[...] = s - m - jnp.log(
        jnp.sum(jnp.exp(s - m), axis=1, keepdims=True))


def kernel(adj, features, W1, b1, W2, b2, W3, b3, W4, b4, att_W, fc1_W,
           fc1_b, sc_W, sc_b):
    n, d = features.shape
    e = adj.shape[1]
    chunk = 80
    f32 = jnp.float32
    epw = e // NW
    src = adj[0].reshape(NW, epw // chunk, chunk)
    dst = adj[1].reshape(NW, epw // chunk, chunk)

    deg_k = _make_degree(n, e, chunk)
    degp = deg_k(dst, jnp.ones((chunk, 16), f32), jnp.zeros((n, 16), f32))

    g1 = _tc_call(
        _k_matmul, jax.ShapeDtypeStruct((n, W1.shape[1]), f32)
    )(features, W1)

    dinv, gs = _tc_call(
        _k_scale,
        (jax.ShapeDtypeStruct((n, 1), f32),
         jax.ShapeDtypeStruct((n, W1.shape[1]), f32)),
    )(degp, g1)

    weights = [(b1, W2), (b2, W3), (b3, W4)]
    for b, wn in weights:
        f = gs.shape[1]
        scat = _make_edge_scatter(n, e, f, chunk)
        p = scat(src, dst, gs, jnp.zeros((n, f), f32))
        gs = _tc_call(
            _k_mid, jax.ShapeDtypeStruct((n, wn.shape[1]), f32)
        )(p, gs, b, dinv, wn)

    f = gs.shape[1]
    scat = _make_edge_scatter(n, e, f, chunk)
    p = scat(src, dst, gs, jnp.zeros((n, f), f32))
    out = _tc_call(
        _k_last, jax.ShapeDtypeStruct((1, sc_W.shape[1]), f32)
    )(p, gs, b4, dinv, att_W, fc1_W, fc1_b, sc_W, sc_b)
    return out


# speedup vs baseline: 49.7344x; 1.1793x over previous
import functools

import jax
import jax.numpy as jnp
from jax import lax
from jax.experimental import pallas as pl
from jax.experimental.pallas import tpu as pltpu
from jax.experimental.pallas import tpu_sc as plsc

NC = 2
NS = 16
NW = NC * NS


def _make_edge_scatter(n, e, f, chunk):
    assert e % NW == 0
    epw = e // NW
    assert epw % chunk == 0 and chunk % 8 == 0 and chunk <= 128
    nchunks = epw // chunk
    nbuf = 5
    nring = 2 * nbuf
    rpt = (n // NS) & ~7
    rem = n - NS * rpt

    mesh = plsc.VectorSubcoreMesh(core_axis_name="c", subcore_axis_name="s")

    @functools.partial(
        pl.kernel,
        out_type=jax.ShapeDtypeStruct((NC, n, f), jnp.float32),
        mesh=mesh,
        scratch_types=[
            pltpu.VMEM((nchunks, chunk), jnp.int32),
            pltpu.VMEM((nchunks, chunk), jnp.int32),
            pltpu.VMEM((nring, chunk, f), jnp.float32),
            pltpu.VMEM_SHARED((n, f), jnp.float32),
            pltpu.SemaphoreType.DMA,
            pltpu.SemaphoreType.DMA,
        ],
        compiler_params=pltpu.CompilerParams(use_tc_tiling_on_sc=False),
    )
    def k(src_hbm, dst_hbm, h_hbm, zero_hbm, out_hbm, sidx, didx, rows, acc,
          gsem, ssem):
        c = lax.axis_index("c")
        s = lax.axis_index("s")
        wid = s * NC + c

        pltpu.sync_copy(src_hbm.at[wid], sidx)
        pltpu.sync_copy(dst_hbm.at[wid], didx)

        pltpu.sync_copy(zero_hbm.at[pl.ds(s * rpt, rpt)],
                        acc.at[pl.ds(s * rpt, rpt)])
        if rem:
            @pl.when(s == NS - 1)
            def _():
                pltpu.sync_copy(zero_hbm.at[pl.ds(NS * rpt, rem)],
                                acc.at[pl.ds(NS * rpt, rem)])
        plsc.subcore_barrier()

        for b in range(nbuf):
            pltpu.async_copy(h_hbm.at[sidx.at[b]], rows.at[b], gsem)

        def chunk_body(j, carry):
            slot = lax.rem(j, nring)
            pltpu.make_async_copy(h_hbm.at[sidx.at[j]], rows.at[slot],
                                  gsem).wait()
            pltpu.async_copy(rows.at[slot], acc.at[didx.at[j]], ssem,
                             add=True)

            @pl.when(j >= nbuf)
            def _():
                pltpu.make_async_copy(rows.at[slot], acc.at[didx.at[j]],
                                      ssem).wait()

            @pl.when(j + nbuf < nchunks)
            def _():
                pltpu.async_copy(h_hbm.at[sidx.at[j + nbuf]],
                                 rows.at[lax.rem(j + nbuf, nring)], gsem)
            return carry

        lax.fori_loop(0, nchunks, chunk_body, 0)
        for b in range(nbuf):
            pltpu.make_async_copy(rows.at[0], acc.at[didx.at[0]],
                                  ssem).wait()
        plsc.subcore_barrier()
        pltpu.sync_copy(acc.at[pl.ds(s * rpt, rpt)],
                        out_hbm.at[c, pl.ds(s * rpt, rpt)])
        if rem:
            @pl.when(s == NS - 1)
            def _():
                pltpu.sync_copy(acc.at[pl.ds(NS * rpt, rem)],
                                out_hbm.at[c, pl.ds(NS * rpt, rem)])

    return k


def _make_degree(n, e, chunk):
    assert e % NW == 0
    epw = e // NW
    assert epw % chunk == 0 and chunk % 8 == 0 and chunk <= 128
    nchunks = epw // chunk
    rpt = (n // NS) & ~7
    rem = n - NS * rpt
    f = 16

    mesh = plsc.VectorSubcoreMesh(core_axis_name="c", subcore_axis_name="s")

    @functools.partial(
        pl.kernel,
        out_type=jax.ShapeDtypeStruct((NC, n, f), jnp.float32),
        mesh=mesh,
        scratch_types=[
            pltpu.VMEM((nchunks, chunk), jnp.int32),
            pltpu.VMEM((chunk, f), jnp.float32),
            pltpu.VMEM_SHARED((n, f), jnp.float32),
            pltpu.SemaphoreType.DMA,
        ],
        compiler_params=pltpu.CompilerParams(use_tc_tiling_on_sc=False),
    )
    def k(dst_hbm, ones_hbm, zero_hbm, out_hbm, didx, ones_v, acc, ssem):
        c = lax.axis_index("c")
        s = lax.axis_index("s")
        wid = s * NC + c

        pltpu.sync_copy(dst_hbm.at[wid], didx)
        pltpu.sync_copy(ones_hbm, ones_v)
        pltpu.sync_copy(zero_hbm.at[pl.ds(s * rpt, rpt)],
                        acc.at[pl.ds(s * rpt, rpt)])
        if rem:
            @pl.when(s == NS - 1)
            def _():
                pltpu.sync_copy(zero_hbm.at[pl.ds(NS * rpt, rem)],
                                acc.at[pl.ds(NS * rpt, rem)])
        plsc.subcore_barrier()

        lag = 8

        def chunk_body(j, carry):
            pltpu.async_copy(ones_v, acc.at[didx.at[j]], ssem, add=True)

            @pl.when(j >= lag)
            def _():
                pltpu.make_async_copy(ones_v, acc.at[didx.at[j]],
                                      ssem).wait()
            return carry

        lax.fori_loop(0, nchunks, chunk_body, 0)
        for _ in range(min(lag, nchunks)):
            pltpu.make_async_copy(ones_v, acc.at[didx.at[0]],
                                  ssem).wait()
        plsc.subcore_barrier()
        pltpu.sync_copy(acc.at[pl.ds(s * rpt, rpt)],
                        out_hbm.at[c, pl.ds(s * rpt, rpt)])
        if rem:
            @pl.when(s == NS - 1)
            def _():
                pltpu.sync_copy(acc.at[pl.ds(NS * rpt, rem)],
                                out_hbm.at[c, pl.ds(NS * rpt, rem)])

    return k


def _tc_call(body, out_shapes):
    return pl.pallas_call(body, out_shape=out_shapes)


def _k_matmul1(x_ref, wbd_ref, g_ref):
    xp = jnp.reshape(x_ref[...], (x_ref.shape[0] // 256, 256))
    g = jnp.dot(xp, wbd_ref[...], preferred_element_type=jnp.float32)
    g_ref[...] = jnp.reshape(g, (-1,))


def _k_scale(degp_ref, g_ref, m64_ref, m32_ref, dinv64_ref, dinv32_ref,
             gs_ref):
    r = degp_ref.shape[0] // 256
    dd = jnp.reshape(degp_ref[...], (2 * r, 128))
    deg8 = dd[:r] + dd[r:]
    dinv8 = lax.rsqrt(1.0 + deg8)
    d64 = jnp.dot(dinv8, m64_ref[...], preferred_element_type=jnp.float32)
    d64 = jnp.reshape(d64, (-1,))
    d32 = jnp.dot(dinv8, m32_ref[...], preferred_element_type=jnp.float32)
    d32 = jnp.reshape(d32, (-1,))
    dinv64_ref[...] = d64
    dinv32_ref[...] = d32
    gs_ref[...] = d64 * g_ref[...]


def _combine(p_ref, gs_ref, dinvf_ref, bb):
    nf = gs_ref.shape[0]
    x = nf // 128
    p0 = jnp.reshape(p_ref[pl.ds(0, nf)], (x, 128))
    p1 = jnp.reshape(p_ref[pl.ds(nf, nf)], (x, 128))
    gs = jnp.reshape(gs_ref[...], (x, 128))
    dv = jnp.reshape(dinvf_ref[...], (x, 128))
    y = dv * (p0 + p1 + gs) + bb[None, :]
    return jnp.maximum(y, 0.0)


def _k_mid(p_ref, gs_ref, bb_ref, dinvf_ref, dinvo_ref, wbd_ref, gsn_ref,
           *, regroup):
    y = _combine(p_ref, gs_ref, dinvf_ref, bb_ref[...])
    if regroup:
        y = jnp.reshape(jnp.reshape(y, (-1,)), (y.shape[0] // 2, 256))
    h = jnp.dot(y, wbd_ref[...], preferred_element_type=jnp.float32)
    gsn_ref[...] = dinvo_ref[...] * jnp.reshape(h, (-1,))


def _k_last(p_ref, gs_ref, bb_ref, dinvf_ref, attbd_ref, g4_ref, g4t_ref,
            gsel_ref, gexp_ref, fc1w_ref, fc1b_ref, scw_ref, scb_ref,
            out_ref):
    h = _combine(p_ref, gs_ref, dinvf_ref, bb_ref[...])
    n = gs_ref.shape[0] // 32
    hw = jnp.dot(h, attbd_ref[...], preferred_element_type=jnp.float32)
    sp = jnp.sum(hw, axis=0)[None, :]
    gc = jnp.dot(sp, gsel_ref[...],
                 preferred_element_type=jnp.float32) * (1.0 / n)
    tg = jnp.tanh(gc)
    tgt = jnp.dot(tg, gexp_ref[...],
                  preferred_element_type=jnp.float32)
    s4 = jnp.dot(h * tgt, g4_ref[...],
                 preferred_element_type=jnp.float32)
    s4 = jax.nn.sigmoid(s4)
    s4e = jnp.dot(s4, g4t_ref[...],
                  preferred_element_type=jnp.float32)
    rep128 = jnp.sum(h * s4e, axis=0)[None, :]
    rep = jnp.dot(rep128, gsel_ref[...],
                  preferred_element_type=jnp.float32)
    s = jnp.dot(rep, fc1w_ref[...], preferred_element_type=jnp.float32)
    s = jnp.maximum(s + fc1b_ref[...][None, :], 0.0)
    s = jnp.dot(s, scw_ref[...], preferred_element_type=jnp.float32)
    s = s + scb_ref[...][None, :]
    m = jnp.max(s, axis=1, keepdims=True)
    out_ref[...] = s - m - jnp.log(
        jnp.sum(jnp.exp(s - m), axis=1, keepdims=True))


def kernel(adj, features, W1, b1, W2, b2, W3, b3, W4, b4, att_W, fc1_W,
           fc1_b, sc_W, sc_b):
    n, d = features.shape
    e = adj.shape[1]
    chunk = 80
    f32 = jnp.float32
    epw = e // NW
    src = adj[0].reshape(NW, epw // chunk, chunk)
    dst = adj[1].reshape(NW, epw // chunk, chunk)

    def bdiag(w, q):
        return jnp.kron(jnp.eye(q, dtype=f32), w)

    lanes = jnp.arange(128)
    m64 = jnp.zeros((128, 512), f32)
    for m in range(4):
        m64 = m64.at[(2 * m + lanes // 64) * 16, m * 128 + lanes].set(1.0)
    m32 = jnp.zeros((128, 256), f32)
    for m in range(2):
        m32 = m32.at[(4 * m + lanes // 32) * 16, m * 128 + lanes].set(1.0)
    g4 = (lanes[:, None] // 32 == jnp.arange(4)[None, :]).astype(f32)
    g4t = g4.T
    gsel = (lanes[:, None] % 32 == jnp.arange(32)[None, :]).astype(f32)
    gexp = gsel.T

    deg_k = _make_degree(n, e, chunk)
    degp = deg_k(dst, jnp.ones((chunk, 16), f32),
                 jnp.zeros((n * 16,), f32).reshape(n, 16))

    g1 = _tc_call(
        _k_matmul1, jax.ShapeDtypeStruct((n * W1.shape[1],), f32)
    )(features.reshape(-1), bdiag(W1, 2))

    dinv64, dinv32, gs = _tc_call(
        _k_scale,
        (jax.ShapeDtypeStruct((n * 64,), f32),
         jax.ShapeDtypeStruct((n * 32,), f32),
         jax.ShapeDtypeStruct((n * W1.shape[1],), f32)),
    )(degp.reshape(-1), g1, m64, m32)

    layers = [
        (b1, bdiag(W2, 2), 64, 64, False),
        (b2, bdiag(W3, 4), 64, 32, True),
        (b3, bdiag(W4, 4), 32, 32, False),
    ]
    dinv_of = {64: dinv64, 32: dinv32}
    for b, wbd, f, fn, regroup in layers:
        scat = _make_edge_scatter(n, e, f, chunk)
        p = scat(src, dst, gs.reshape(n, f),
                 jnp.zeros((n * f,), f32).reshape(n, f))
        gs = _tc_call(
            functools.partial(_k_mid, regroup=regroup),
            jax.ShapeDtypeStruct((n * fn,), f32),
        )(p.reshape(-1), gs, jnp.tile(b, 128 // f), dinv_of[f],
          dinv_of[fn], wbd)

    f = b4.shape[0]
    scat = _make_edge_scatter(n, e, f, chunk)
    p = scat(src, dst, gs.reshape(n, f),
             jnp.zeros((n * f,), f32).reshape(n, f))
    out = _tc_call(
        _k_last, jax.ShapeDtypeStruct((1, sc_W.shape[1]), f32)
    )(p.reshape(-1), gs, jnp.tile(b4, 128 // f), dinv_of[f],
      bdiag(att_W, 4), g4, g4t, gsel, gexp, fc1_W, fc1_b, sc_W, sc_b)
    return out
